# baseline (speedup 1.0000x reference)
import functools

import jax
import jax.numpy as jnp
from jax import lax
from jax.experimental import pallas as pl
from jax.experimental.pallas import tpu as pltpu
from jax.experimental.pallas import tpu_sc as plsc

S = 8192
B = 4096
T = 50
T1 = T - 1
NC = 2
NS = 16
NW = NC * NS
PW = B // NW
L = 16
RBLK = 256
R_TC = 5632
N_SC = S - R_TC
P_SC = N_SC // NW
G_SC = P_SC // 8


def _tc_lse_body(t_ref, init_ref, rowlse_ref, initlse_ref):
    i = pl.program_id(0)
    blk = t_ref[...]
    m = jnp.max(blk, axis=1)
    ssum = jnp.sum(jnp.exp(blk - m[:, None]), axis=1)
    rowlse_ref[...] = m + jnp.log(ssum)

    @pl.when(i == 0)
    def _():
        v = init_ref[...]
        mi = jnp.max(v)
        lse0 = mi + jnp.log(jnp.sum(jnp.exp(v - mi)))
        initlse_ref[...] = jnp.full((8, 128), lse0, dtype=jnp.float32)


def _tc_lse(t_mat, init_vec):
    return pl.pallas_call(
        _tc_lse_body,
        grid=(R_TC // RBLK,),
        in_specs=[
            pl.BlockSpec((RBLK, S), lambda i: (i, 0)),
            pl.BlockSpec((S,), lambda i: (0,)),
        ],
        out_specs=[
            pl.BlockSpec((RBLK,), lambda i: (i,)),
            pl.BlockSpec((8, 128), lambda i: (0, 0)),
        ],
        out_shape=[
            jax.ShapeDtypeStruct((R_TC,), jnp.float32),
            jax.ShapeDtypeStruct((8, 128), jnp.float32),
        ],
        compiler_params=pltpu.CompilerParams(
            dimension_semantics=("arbitrary",),
        ),
    )(t_mat, init_vec)


def _tc_fixup_body(lse_tc_ref, s_sc_ref, out_ref):
    out_ref[pl.ds(0, R_TC)] = lse_tc_ref[...]
    out_ref[pl.ds(R_TC, N_SC)] = jnp.log(s_sc_ref[...])


def _tc_fixup(lse_tc, s_sc):
    return pl.pallas_call(
        _tc_fixup_body,
        out_shape=jax.ShapeDtypeStruct((S,), jnp.float32),
    )(lse_tc, s_sc)


def _sc_rowsum_body(t4_hbm, s_hbm, buf_v, out_v, sems):
    wid = lax.axis_index("s") * NC + lax.axis_index("c")
    rt0 = R_TC // 8 + wid * G_SC

    nch = G_SC * 2
    hct = (S // 128) // 2

    def issue(i):
        g, half = divmod(i, 2)
        return pltpu.async_copy(
            t4_hbm.at[rt0 + g, pl.ds(half * hct, hct)], buf_v.at[i % 2],
            sems.at[i % 2])

    lanes = lax.iota(jnp.int32, L)
    copies = {0: issue(0)}
    rowvec = jnp.zeros((L,), jnp.float32)
    accs = None
    for i in range(nch):
        g, half = divmod(i, 2)
        if i + 1 < nch:
            copies[i + 1] = issue(i + 1)
        copies[i].wait()
        if half == 0:
            accs = tuple(jnp.zeros((L,), jnp.float32) for _ in range(8))

        def ct_body(ct, a8, _i=i):
            new = []
            for sl in range(8):
                a = a8[sl]
                for k in range(8):
                    a = a + jnp.exp(buf_v[_i % 2, ct, sl, pl.ds(k * L, L)])
                new.append(a)
            return tuple(new)

        accs = lax.fori_loop(0, hct, ct_body, accs)
        if half == 1:
            for sl in range(8):
                rowvec = jnp.where(lanes == (g % 2) * 8 + sl,
                                   jnp.sum(accs[sl]), rowvec)
            if g % 2 == 1:
                out_v[pl.ds((g // 2) * L, L)] = rowvec

    pltpu.sync_copy(out_v, s_hbm.at[pl.ds(wid * P_SC, P_SC)])


@functools.cache
def _sc_rowsum():
    return pl.kernel(
        _sc_rowsum_body,
        out_type=jax.ShapeDtypeStruct((N_SC,), jnp.float32),
        mesh=_sc_mesh(),
        scratch_types=[
            pltpu.VMEM((2, (S // 128) // 2, 8, 128), jnp.float32),
            pltpu.VMEM((P_SC,), jnp.float32),
            pltpu.SemaphoreType.DMA((2,)),
        ],
        compiler_params=pltpu.CompilerParams(needs_layout_passes=False),
    )


def _sc_gather_body(prev_hbm, cur_hbm, first_hbm, tflat_hbm, init_hbm,
                    part_hbm,
                    prev_v, cur_v, idx_v, vals_v, init_v, first_v, acc_v, sem):
    wid = lax.axis_index("s") * NC + lax.axis_index("c")
    base = wid * PW

    pltpu.sync_copy(prev_hbm.at[wid], prev_v)
    pltpu.sync_copy(cur_hbm.at[wid], cur_v)
    pltpu.sync_copy(first_hbm.at[pl.ds(base, PW)], first_v)
    pltpu.sync_copy(init_hbm, init_v)

    def idx_body(j, carry):
        for cc in range(PW // L):
            p = prev_v[j, pl.ds(cc * L, L)]
            q = cur_v[j, pl.ds(cc * L, L)]
            idx_v[j, pl.ds(cc * L, L)] = (
                ((p >> 3) << 16) + ((q >> 7) << 10) + ((p & 7) << 7)
                + (q & 127))
        return carry

    lax.fori_loop(0, T1, idx_body, 0)

    def gather_group(g, carry):
        copies = []
        for u in range(7):
            j = g * 7 + u
            copies.append(
                pltpu.async_copy(tflat_hbm.at[idx_v.at[j]], vals_v.at[j], sem))
        for cp in copies:
            cp.wait()
        return carry

    lax.fori_loop(0, 7, gather_group, 0)

    for cc in range(PW // L):
        sl = pl.ds(cc * L, L)

        def acc_body(j, acc):
            return acc + vals_v[j, sl]

        acc16 = lax.fori_loop(0, T1, acc_body, jnp.zeros((L,), jnp.float32))
        acc_v[sl] = acc16 + plsc.load_gather(init_v, [first_v[sl]])

    pltpu.sync_copy(acc_v, part_hbm.at[pl.ds(base, PW)])


def _sc_combine_body(prev_hbm, lse0_hbm, part_hbm, rowlse_hbm, out_hbm,
                     prev_v, lse_v, part_v, lse0_v, acc_v, sem):
    wid = lax.axis_index("s") * NC + lax.axis_index("c")
    base = wid * PW

    pltpu.sync_copy(prev_hbm.at[wid], prev_v)
    pltpu.sync_copy(part_hbm.at[pl.ds(base, PW)], part_v)
    pltpu.sync_copy(lse0_hbm, lse0_v)
    pltpu.sync_copy(rowlse_hbm, lse_v)

    for cc in range(PW // L):
        sl = pl.ds(cc * L, L)

        def acc_body(j, acc):
            return acc + plsc.load_gather(lse_v, [prev_v[j, sl]])

        acc16 = lax.fori_loop(0, T1, acc_body, jnp.zeros((L,), jnp.float32))
        acc_v[sl] = part_v[sl] - acc16 - lse0_v[...]

    pltpu.sync_copy(acc_v, out_hbm.at[pl.ds(base, PW)])


def _sc_mesh():
    return plsc.VectorSubcoreMesh(
        core_axis_name="c", subcore_axis_name="s", num_cores=NC,
        num_subcores=NS)


@functools.cache
def _sc_gather():
    return pl.kernel(
        _sc_gather_body,
        out_type=jax.ShapeDtypeStruct((B,), jnp.float32),
        mesh=_sc_mesh(),
        scratch_types=[
            pltpu.VMEM((T1, PW), jnp.int32),
            pltpu.VMEM((T1, PW), jnp.int32),
            pltpu.VMEM((T1, PW), jnp.int32),
            pltpu.VMEM((T1, PW), jnp.float32),
            pltpu.VMEM((S,), jnp.float32),
            pltpu.VMEM((PW,), jnp.int32),
            pltpu.VMEM((PW,), jnp.float32),
            pltpu.SemaphoreType.DMA,
        ],
        compiler_params=pltpu.CompilerParams(needs_layout_passes=False),
    )


@functools.cache
def _sc_combine():
    return pl.kernel(
        _sc_combine_body,
        out_type=jax.ShapeDtypeStruct((B,), jnp.float32),
        mesh=_sc_mesh(),
        scratch_types=[
            pltpu.VMEM((T1, PW), jnp.int32),
            pltpu.VMEM((S,), jnp.float32),
            pltpu.VMEM((PW,), jnp.float32),
            pltpu.VMEM((L,), jnp.float32),
            pltpu.VMEM((PW,), jnp.float32),
            pltpu.SemaphoreType.DMA,
        ],
        compiler_params=pltpu.CompilerParams(needs_layout_passes=False),
    )


def kernel(x, initial_state_vector, state_transition_matrix):
    x = x.astype(jnp.int32)
    row_lse, init_lse = _tc_lse(state_transition_matrix, initial_state_vector)

    xt = x.T
    prev_w = xt[:-1].reshape(T1, NW, PW).transpose(1, 0, 2)
    cur_w = xt[1:].reshape(T1, NW, PW).transpose(1, 0, 2)
    first = x[:, 0]
    lse0_vec = jnp.full((L,), init_lse[0, 0], dtype=jnp.float32)
    t4 = state_transition_matrix.reshape(
        S // 8, 8, S // 128, 128).transpose(0, 2, 1, 3)
    t_flat = t4.reshape(-1)

    s_sc = _sc_rowsum()(t4)
    partial = _sc_gather()(prev_w, cur_w, first, t_flat,
                           initial_state_vector)
    row_lse_full = _tc_fixup(row_lse, s_sc)
    return _sc_combine()(prev_w, lse0_vec, partial, row_lse_full)


# speedup vs baseline: 1.0212x; 1.0212x over previous
import functools

import jax
import jax.numpy as jnp
from jax import lax
from jax.experimental import pallas as pl
from jax.experimental.pallas import tpu as pltpu
from jax.experimental.pallas import tpu_sc as plsc

S = 8192
B = 4096
T = 50
T1 = T - 1
NC = 2
NS = 16
NW = NC * NS
PW = B // NW
L = 16
RBLK = 256
R_TC = 6912
N_SC = S - R_TC
P_SC = N_SC // NW
G_SC = P_SC // 8


def _tc_lse_body(t_ref, init_ref, rowlse_ref, initlse_ref):
    i = pl.program_id(0)
    blk = t_ref[...]
    m = jnp.max(blk, axis=1)
    ssum = jnp.sum(jnp.exp(blk - m[:, None]), axis=1)
    rowlse_ref[...] = m + jnp.log(ssum)

    @pl.when(i == 0)
    def _():
        v = init_ref[...]
        mi = jnp.max(v)
        lse0 = mi + jnp.log(jnp.sum(jnp.exp(v - mi)))
        initlse_ref[...] = jnp.full((8, 128), lse0, dtype=jnp.float32)


def _tc_lse(t_mat, init_vec):
    return pl.pallas_call(
        _tc_lse_body,
        grid=(R_TC // RBLK,),
        in_specs=[
            pl.BlockSpec((RBLK, S), lambda i: (i, 0)),
            pl.BlockSpec((S,), lambda i: (0,)),
        ],
        out_specs=[
            pl.BlockSpec((RBLK,), lambda i: (i,)),
            pl.BlockSpec((8, 128), lambda i: (0, 0)),
        ],
        out_shape=[
            jax.ShapeDtypeStruct((R_TC,), jnp.float32),
            jax.ShapeDtypeStruct((8, 128), jnp.float32),
        ],
        compiler_params=pltpu.CompilerParams(
            dimension_semantics=("arbitrary",),
        ),
    )(t_mat, init_vec)


def _tc_fixup_body(lse_tc_ref, s_sc_ref, out_ref):
    out_ref[pl.ds(0, R_TC)] = lse_tc_ref[...]
    out_ref[pl.ds(R_TC, N_SC)] = jnp.log(s_sc_ref[...])


def _tc_fixup(lse_tc, s_sc):
    return pl.pallas_call(
        _tc_fixup_body,
        out_shape=jax.ShapeDtypeStruct((S,), jnp.float32),
    )(lse_tc, s_sc)


def _sc_rowsum_body(t4_hbm, s_hbm, buf_v, out_v, sems):
    wid = lax.axis_index("s") * NC + lax.axis_index("c")
    rt0 = R_TC // 8 + wid * G_SC

    nch = G_SC * 2
    hct = (S // 128) // 2

    def issue(i):
        g, half = divmod(i, 2)
        return pltpu.async_copy(
            t4_hbm.at[rt0 + g, pl.ds(half * hct, hct)], buf_v.at[i % 2],
            sems.at[i % 2])

    lanes = lax.iota(jnp.int32, L)
    copies = {0: issue(0)}
    rowvec = jnp.zeros((L,), jnp.float32)
    accs = None
    for i in range(nch):
        g, half = divmod(i, 2)
        if i + 1 < nch:
            copies[i + 1] = issue(i + 1)
        copies[i].wait()
        if half == 0:
            accs = tuple(jnp.zeros((L,), jnp.float32) for _ in range(8))

        def ct_body(ct, a8, _i=i):
            new = []
            for sl in range(8):
                a = a8[sl]
                for k in range(8):
                    a = a + jnp.exp(buf_v[_i % 2, ct, sl, pl.ds(k * L, L)])
                new.append(a)
            return tuple(new)

        accs = lax.fori_loop(0, hct, ct_body, accs)
        if half == 1:
            for sl in range(8):
                rowvec = jnp.where(lanes == (g % 2) * 8 + sl,
                                   jnp.sum(accs[sl]), rowvec)
            if g % 2 == 1:
                out_v[pl.ds((g // 2) * L, L)] = rowvec

    pltpu.sync_copy(out_v, s_hbm.at[pl.ds(wid * P_SC, P_SC)])


@functools.cache
def _sc_rowsum():
    return pl.kernel(
        _sc_rowsum_body,
        out_type=jax.ShapeDtypeStruct((N_SC,), jnp.float32),
        mesh=_sc_mesh(),
        scratch_types=[
            pltpu.VMEM((2, (S // 128) // 2, 8, 128), jnp.float32),
            pltpu.VMEM((P_SC,), jnp.float32),
            pltpu.SemaphoreType.DMA((2,)),
        ],
        compiler_params=pltpu.CompilerParams(needs_layout_passes=False),
    )


def _sc_gather_body(prev_hbm, cur_hbm, first_hbm, tflat_hbm, init_hbm,
                    part_hbm,
                    prev_v, cur_v, idx_v, vals_v, init_v, first_v, acc_v, sem):
    wid = lax.axis_index("s") * NC + lax.axis_index("c")
    base = wid * PW

    pltpu.sync_copy(prev_hbm.at[wid], prev_v)
    pltpu.sync_copy(cur_hbm.at[wid], cur_v)
    pltpu.sync_copy(first_hbm.at[pl.ds(base, PW)], first_v)
    pltpu.sync_copy(init_hbm, init_v)

    def idx_body(j, carry):
        for cc in range(PW // L):
            p = prev_v[j, pl.ds(cc * L, L)]
            q = cur_v[j, pl.ds(cc * L, L)]
            idx_v[j, pl.ds(cc * L, L)] = (
                ((p >> 3) << 16) + ((q >> 7) << 10) + ((p & 7) << 7)
                + (q & 127))
        return carry

    lax.fori_loop(0, T1, idx_body, 0)

    def gather_group(g, carry):
        copies = []
        for u in range(7):
            j = g * 7 + u
            copies.append(
                pltpu.async_copy(tflat_hbm.at[idx_v.at[j]], vals_v.at[j], sem))
        for cp in copies:
            cp.wait()
        return carry

    lax.fori_loop(0, 7, gather_group, 0)

    for cc in range(PW // L):
        sl = pl.ds(cc * L, L)

        def acc_body(j, acc):
            return acc + vals_v[j, sl]

        acc16 = lax.fori_loop(0, T1, acc_body, jnp.zeros((L,), jnp.float32))
        acc_v[sl] = acc16 + plsc.load_gather(init_v, [first_v[sl]])

    pltpu.sync_copy(acc_v, part_hbm.at[pl.ds(base, PW)])


def _sc_combine_body(prev_hbm, lse0_hbm, part_hbm, rowlse_hbm, out_hbm,
                     prev_v, lse_v, part_v, lse0_v, acc_v, sem):
    wid = lax.axis_index("s") * NC + lax.axis_index("c")
    base = wid * PW

    pltpu.sync_copy(prev_hbm.at[wid], prev_v)
    pltpu.sync_copy(part_hbm.at[pl.ds(base, PW)], part_v)
    pltpu.sync_copy(lse0_hbm, lse0_v)
    pltpu.sync_copy(rowlse_hbm, lse_v)

    for cc in range(PW // L):
        sl = pl.ds(cc * L, L)

        def acc_body(j, acc):
            return acc + plsc.load_gather(lse_v, [prev_v[j, sl]])

        acc16 = lax.fori_loop(0, T1, acc_body, jnp.zeros((L,), jnp.float32))
        acc_v[sl] = part_v[sl] - acc16 - lse0_v[...]

    pltpu.sync_copy(acc_v, out_hbm.at[pl.ds(base, PW)])


def _sc_mesh():
    return plsc.VectorSubcoreMesh(
        core_axis_name="c", subcore_axis_name="s", num_cores=NC,
        num_subcores=NS)


@functools.cache
def _sc_gather():
    return pl.kernel(
        _sc_gather_body,
        out_type=jax.ShapeDtypeStruct((B,), jnp.float32),
        mesh=_sc_mesh(),
        scratch_types=[
            pltpu.VMEM((T1, PW), jnp.int32),
            pltpu.VMEM((T1, PW), jnp.int32),
            pltpu.VMEM((T1, PW), jnp.int32),
            pltpu.VMEM((T1, PW), jnp.float32),
            pltpu.VMEM((S,), jnp.float32),
            pltpu.VMEM((PW,), jnp.int32),
            pltpu.VMEM((PW,), jnp.float32),
            pltpu.SemaphoreType.DMA,
        ],
        compiler_params=pltpu.CompilerParams(needs_layout_passes=False),
    )


@functools.cache
def _sc_combine():
    return pl.kernel(
        _sc_combine_body,
        out_type=jax.ShapeDtypeStruct((B,), jnp.float32),
        mesh=_sc_mesh(),
        scratch_types=[
            pltpu.VMEM((T1, PW), jnp.int32),
            pltpu.VMEM((S,), jnp.float32),
            pltpu.VMEM((PW,), jnp.float32),
            pltpu.VMEM((L,), jnp.float32),
            pltpu.VMEM((PW,), jnp.float32),
            pltpu.SemaphoreType.DMA,
        ],
        compiler_params=pltpu.CompilerParams(needs_layout_passes=False),
    )


def kernel(x, initial_state_vector, state_transition_matrix):
    x = x.astype(jnp.int32)
    row_lse, init_lse = _tc_lse(state_transition_matrix, initial_state_vector)

    xt = x.T
    prev_w = xt[:-1].reshape(T1, NW, PW).transpose(1, 0, 2)
    cur_w = xt[1:].reshape(T1, NW, PW).transpose(1, 0, 2)
    first = x[:, 0]
    lse0_vec = jnp.full((L,), init_lse[0, 0], dtype=jnp.float32)
    t4 = state_transition_matrix.reshape(
        S // 8, 8, S // 128, 128).transpose(0, 2, 1, 3)
    t_flat = t4.reshape(-1)

    s_sc = _sc_rowsum()(t4)
    partial = _sc_gather()(prev_w, cur_w, first, t_flat,
                           initial_state_vector)
    row_lse_full = _tc_fixup(row_lse, s_sc)
    return _sc_combine()(prev_w, lse0_vec, partial, row_lse_full)


# speedup vs baseline: 1.0415x; 1.0199x over previous
import functools

import jax
import jax.numpy as jnp
from jax import lax
from jax.experimental import pallas as pl
from jax.experimental.pallas import tpu as pltpu
from jax.experimental.pallas import tpu_sc as plsc

S = 8192
B = 4096
T = 50
T1 = T - 1
NC = 2
NS = 16
NW = NC * NS
PW = B // NW
L = 16
RBLK = 512
R_TC = 7168
N_SC = S - R_TC
P_SC = N_SC // NW
G_SC = P_SC // 8


def _tc_lse_body(t_ref, init_ref, rowlse_ref, initlse_ref):
    i = pl.program_id(0)
    blk = t_ref[...]
    m = jnp.max(blk, axis=1)
    ssum = jnp.sum(jnp.exp(blk - m[:, None]), axis=1)
    rowlse_ref[...] = m + jnp.log(ssum)

    @pl.when(i == 0)
    def _():
        v = init_ref[...]
        mi = jnp.max(v)
        lse0 = mi + jnp.log(jnp.sum(jnp.exp(v - mi)))
        initlse_ref[...] = jnp.full((8, 128), lse0, dtype=jnp.float32)


def _tc_lse(t_mat, init_vec):
    return pl.pallas_call(
        _tc_lse_body,
        grid=(R_TC // RBLK,),
        in_specs=[
            pl.BlockSpec((RBLK, S), lambda i: (i, 0)),
            pl.BlockSpec((S,), lambda i: (0,)),
        ],
        out_specs=[
            pl.BlockSpec((RBLK,), lambda i: (i,)),
            pl.BlockSpec((8, 128), lambda i: (0, 0)),
        ],
        out_shape=[
            jax.ShapeDtypeStruct((R_TC,), jnp.float32),
            jax.ShapeDtypeStruct((8, 128), jnp.float32),
        ],
        compiler_params=pltpu.CompilerParams(
            dimension_semantics=("arbitrary",),
        ),
    )(t_mat, init_vec)


def _tc_fixup_body(lse_tc_ref, s_sc_ref, out_ref):
    out_ref[pl.ds(0, R_TC)] = lse_tc_ref[...]
    out_ref[pl.ds(R_TC, N_SC)] = jnp.log(s_sc_ref[...])


def _tc_fixup(lse_tc, s_sc):
    return pl.pallas_call(
        _tc_fixup_body,
        out_shape=jax.ShapeDtypeStruct((S,), jnp.float32),
    )(lse_tc, s_sc)


def _sc_rowsum_body(t4_hbm, s_hbm, buf_v, out_v, sems):
    wid = lax.axis_index("s") * NC + lax.axis_index("c")
    rt0 = R_TC // 8 + wid * G_SC

    nch = G_SC * 2
    hct = (S // 128) // 2

    def issue(i):
        g, half = divmod(i, 2)
        return pltpu.async_copy(
            t4_hbm.at[rt0 + g, pl.ds(half * hct, hct)], buf_v.at[i % 2],
            sems.at[i % 2])

    lanes = lax.iota(jnp.int32, L)
    copies = {0: issue(0)}
    rowvec = jnp.zeros((L,), jnp.float32)
    accs = None
    for i in range(nch):
        g, half = divmod(i, 2)
        if i + 1 < nch:
            copies[i + 1] = issue(i + 1)
        copies[i].wait()
        if half == 0:
            accs = tuple(jnp.zeros((L,), jnp.float32) for _ in range(8))

        def ct_body(ct, a8, _i=i):
            new = []
            for sl in range(8):
                a = a8[sl]
                for k in range(8):
                    a = a + jnp.exp(buf_v[_i % 2, ct, sl, pl.ds(k * L, L)])
                new.append(a)
            return tuple(new)

        accs = lax.fori_loop(0, hct, ct_body, accs)
        if half == 1:
            for sl in range(8):
                rowvec = jnp.where(lanes == (g % 2) * 8 + sl,
                                   jnp.sum(accs[sl]), rowvec)
            if g % 2 == 1:
                out_v[pl.ds((g // 2) * L, L)] = rowvec

    pltpu.sync_copy(out_v, s_hbm.at[pl.ds(wid * P_SC, P_SC)])


@functools.cache
def _sc_rowsum():
    return pl.kernel(
        _sc_rowsum_body,
        out_type=jax.ShapeDtypeStruct((N_SC,), jnp.float32),
        mesh=_sc_mesh(),
        scratch_types=[
            pltpu.VMEM((2, (S // 128) // 2, 8, 128), jnp.float32),
            pltpu.VMEM((P_SC,), jnp.float32),
            pltpu.SemaphoreType.DMA((2,)),
        ],
        compiler_params=pltpu.CompilerParams(needs_layout_passes=False),
    )


def _sc_gather_body(prev_hbm, cur_hbm, first_hbm, tflat_hbm, init_hbm,
                    part_hbm,
                    prev_v, cur_v, idx_v, vals_v, init_v, first_v, acc_v, sem):
    wid = lax.axis_index("s") * NC + lax.axis_index("c")
    base = wid * PW

    pltpu.sync_copy(prev_hbm.at[wid], prev_v)
    pltpu.sync_copy(cur_hbm.at[wid], cur_v)
    pltpu.sync_copy(first_hbm.at[pl.ds(base, PW)], first_v)
    pltpu.sync_copy(init_hbm, init_v)

    def idx_body(j, carry):
        for cc in range(PW // L):
            p = prev_v[j, pl.ds(cc * L, L)]
            q = cur_v[j, pl.ds(cc * L, L)]
            idx_v[j, pl.ds(cc * L, L)] = (
                ((p >> 3) << 16) + ((q >> 7) << 10) + ((p & 7) << 7)
                + (q & 127))
        return carry

    lax.fori_loop(0, T1, idx_body, 0)

    def gather_group(g, carry):
        copies = []
        for u in range(7):
            j = g * 7 + u
            copies.append(
                pltpu.async_copy(tflat_hbm.at[idx_v.at[j]], vals_v.at[j], sem))
        for cp in copies:
            cp.wait()
        return carry

    lax.fori_loop(0, 7, gather_group, 0)

    for cc in range(PW // L):
        sl = pl.ds(cc * L, L)

        def acc_body(j, acc):
            return acc + vals_v[j, sl]

        acc16 = lax.fori_loop(0, T1, acc_body, jnp.zeros((L,), jnp.float32))
        acc_v[sl] = acc16 + plsc.load_gather(init_v, [first_v[sl]])

    pltpu.sync_copy(acc_v, part_hbm.at[pl.ds(base, PW)])


def _sc_combine_body(prev_hbm, lse0_hbm, part_hbm, rowlse_hbm, out_hbm,
                     prev_v, lse_v, part_v, lse0_v, acc_v, sem):
    wid = lax.axis_index("s") * NC + lax.axis_index("c")
    base = wid * PW

    pltpu.sync_copy(prev_hbm.at[wid], prev_v)
    pltpu.sync_copy(part_hbm.at[pl.ds(base, PW)], part_v)
    pltpu.sync_copy(lse0_hbm, lse0_v)
    pltpu.sync_copy(rowlse_hbm, lse_v)

    for cc in range(PW // L):
        sl = pl.ds(cc * L, L)

        def acc_body(j, acc):
            return acc + plsc.load_gather(lse_v, [prev_v[j, sl]])

        acc16 = lax.fori_loop(0, T1, acc_body, jnp.zeros((L,), jnp.float32))
        acc_v[sl] = part_v[sl] - acc16 - lse0_v[...]

    pltpu.sync_copy(acc_v, out_hbm.at[pl.ds(base, PW)])


def _sc_mesh():
    return plsc.VectorSubcoreMesh(
        core_axis_name="c", subcore_axis_name="s", num_cores=NC,
        num_subcores=NS)


@functools.cache
def _sc_gather():
    return pl.kernel(
        _sc_gather_body,
        out_type=jax.ShapeDtypeStruct((B,), jnp.float32),
        mesh=_sc_mesh(),
        scratch_types=[
            pltpu.VMEM((T1, PW), jnp.int32),
            pltpu.VMEM((T1, PW), jnp.int32),
            pltpu.VMEM((T1, PW), jnp.int32),
            pltpu.VMEM((T1, PW), jnp.float32),
            pltpu.VMEM((S,), jnp.float32),
            pltpu.VMEM((PW,), jnp.int32),
            pltpu.VMEM((PW,), jnp.float32),
            pltpu.SemaphoreType.DMA,
        ],
        compiler_params=pltpu.CompilerParams(needs_layout_passes=False),
    )


@functools.cache
def _sc_combine():
    return pl.kernel(
        _sc_combine_body,
        out_type=jax.ShapeDtypeStruct((B,), jnp.float32),
        mesh=_sc_mesh(),
        scratch_types=[
            pltpu.VMEM((T1, PW), jnp.int32),
            pltpu.VMEM((S,), jnp.float32),
            pltpu.VMEM((PW,), jnp.float32),
            pltpu.VMEM((L,), jnp.float32),
            pltpu.VMEM((PW,), jnp.float32),
            pltpu.SemaphoreType.DMA,
        ],
        compiler_params=pltpu.CompilerParams(needs_layout_passes=False),
    )


def kernel(x, initial_state_vector, state_transition_matrix):
    x = x.astype(jnp.int32)
    row_lse, init_lse = _tc_lse(state_transition_matrix, initial_state_vector)

    xt = x.T
    prev_w = xt[:-1].reshape(T1, NW, PW).transpose(1, 0, 2)
    cur_w = xt[1:].reshape(T1, NW, PW).transpose(1, 0, 2)
    first = x[:, 0]
    lse0_vec = jnp.full((L,), init_lse[0, 0], dtype=jnp.float32)
    t4 = state_transition_matrix.reshape(
        S // 8, 8, S // 128, 128).transpose(0, 2, 1, 3)
    t_flat = t4.reshape(-1)

    s_sc = _sc_rowsum()(t4)
    partial = _sc_gather()(prev_w, cur_w, first, t_flat,
                           initial_state_vector)
    row_lse_full = _tc_fixup(row_lse, s_sc)
    return _sc_combine()(prev_w, lse0_vec, partial, row_lse_full)


# speedup vs baseline: 1.0457x; 1.0040x over previous
import functools

import jax
import jax.numpy as jnp
from jax import lax
from jax.experimental import pallas as pl
from jax.experimental.pallas import tpu as pltpu
from jax.experimental.pallas import tpu_sc as plsc

S = 8192
B = 4096
T = 50
T1 = T - 1
NC = 2
NS = 16
NW = NC * NS
PW = B // NW
L = 16
RBLK = 512
R_TC = 8192
N_SC = S - R_TC
P_SC = N_SC // NW
G_SC = P_SC // 8


def _tc_lse_body(t_ref, init_ref, rowlse_ref, initlse_ref):
    i = pl.program_id(0)
    blk = t_ref[...]
    m = jnp.max(blk, axis=1)
    ssum = jnp.sum(jnp.exp(blk - m[:, None]), axis=1)
    rowlse_ref[...] = m + jnp.log(ssum)

    @pl.when(i == 0)
    def _():
        v = init_ref[...]
        mi = jnp.max(v)
        lse0 = mi + jnp.log(jnp.sum(jnp.exp(v - mi)))
        initlse_ref[...] = jnp.full((8, 128), lse0, dtype=jnp.float32)


def _tc_lse(t_mat, init_vec):
    return pl.pallas_call(
        _tc_lse_body,
        grid=(R_TC // RBLK,),
        in_specs=[
            pl.BlockSpec((RBLK, S), lambda i: (i, 0)),
            pl.BlockSpec((S,), lambda i: (0,)),
        ],
        out_specs=[
            pl.BlockSpec((RBLK,), lambda i: (i,)),
            pl.BlockSpec((8, 128), lambda i: (0, 0)),
        ],
        out_shape=[
            jax.ShapeDtypeStruct((R_TC,), jnp.float32),
            jax.ShapeDtypeStruct((8, 128), jnp.float32),
        ],
        compiler_params=pltpu.CompilerParams(
            dimension_semantics=("arbitrary",),
        ),
    )(t_mat, init_vec)


def _tc_fixup_body(lse_tc_ref, s_sc_ref, out_ref):
    out_ref[pl.ds(0, R_TC)] = lse_tc_ref[...]
    out_ref[pl.ds(R_TC, N_SC)] = jnp.log(s_sc_ref[...])


def _tc_fixup(lse_tc, s_sc):
    return pl.pallas_call(
        _tc_fixup_body,
        out_shape=jax.ShapeDtypeStruct((S,), jnp.float32),
    )(lse_tc, s_sc)


def _sc_rowsum_body(t4_hbm, s_hbm, buf_v, out_v, sems):
    wid = lax.axis_index("s") * NC + lax.axis_index("c")
    rt0 = R_TC // 8 + wid * G_SC

    nch = G_SC * 2
    hct = (S // 128) // 2

    def issue(i):
        g, half = divmod(i, 2)
        return pltpu.async_copy(
            t4_hbm.at[rt0 + g, pl.ds(half * hct, hct)], buf_v.at[i % 2],
            sems.at[i % 2])

    lanes = lax.iota(jnp.int32, L)
    copies = {0: issue(0)}
    rowvec = jnp.zeros((L,), jnp.float32)
    accs = None
    for i in range(nch):
        g, half = divmod(i, 2)
        if i + 1 < nch:
            copies[i + 1] = issue(i + 1)
        copies[i].wait()
        if half == 0:
            accs = tuple(jnp.zeros((L,), jnp.float32) for _ in range(8))

        def ct_body(ct, a8, _i=i):
            new = []
            for sl in range(8):
                a = a8[sl]
                for k in range(8):
                    a = a + jnp.exp(buf_v[_i % 2, ct, sl, pl.ds(k * L, L)])
                new.append(a)
            return tuple(new)

        accs = lax.fori_loop(0, hct, ct_body, accs)
        if half == 1:
            for sl in range(8):
                rowvec = jnp.where(lanes == (g % 2) * 8 + sl,
                                   jnp.sum(accs[sl]), rowvec)
            if g % 2 == 1:
                out_v[pl.ds((g // 2) * L, L)] = rowvec

    pltpu.sync_copy(out_v, s_hbm.at[pl.ds(wid * P_SC, P_SC)])


@functools.cache
def _sc_rowsum():
    return pl.kernel(
        _sc_rowsum_body,
        out_type=jax.ShapeDtypeStruct((N_SC,), jnp.float32),
        mesh=_sc_mesh(),
        scratch_types=[
            pltpu.VMEM((2, (S // 128) // 2, 8, 128), jnp.float32),
            pltpu.VMEM((P_SC,), jnp.float32),
            pltpu.SemaphoreType.DMA((2,)),
        ],
        compiler_params=pltpu.CompilerParams(needs_layout_passes=False),
    )


def _sc_gather_body(prev_hbm, cur_hbm, first_hbm, tflat_hbm, init_hbm,
                    part_hbm,
                    prev_v, cur_v, idx_v, vals_v, init_v, first_v, acc_v, sem):
    wid = lax.axis_index("s") * NC + lax.axis_index("c")
    base = wid * PW

    pltpu.sync_copy(prev_hbm.at[wid], prev_v)
    pltpu.sync_copy(cur_hbm.at[wid], cur_v)
    pltpu.sync_copy(first_hbm.at[pl.ds(base, PW)], first_v)
    pltpu.sync_copy(init_hbm, init_v)

    def idx_body(j, carry):
        for cc in range(PW // L):
            p = prev_v[j, pl.ds(cc * L, L)]
            q = cur_v[j, pl.ds(cc * L, L)]
            idx_v[j, pl.ds(cc * L, L)] = (
                ((p >> 3) << 16) + ((q >> 7) << 10) + ((p & 7) << 7)
                + (q & 127))
        return carry

    lax.fori_loop(0, T1, idx_body, 0)

    def gather_group(g, carry):
        copies = []
        for u in range(7):
            j = g * 7 + u
            copies.append(
                pltpu.async_copy(tflat_hbm.at[idx_v.at[j]], vals_v.at[j], sem))
        for cp in copies:
            cp.wait()
        return carry

    lax.fori_loop(0, 7, gather_group, 0)

    for cc in range(PW // L):
        sl = pl.ds(cc * L, L)

        def acc_body(j, acc):
            return acc + vals_v[j, sl]

        acc16 = lax.fori_loop(0, T1, acc_body, jnp.zeros((L,), jnp.float32))
        acc_v[sl] = acc16 + plsc.load_gather(init_v, [first_v[sl]])

    pltpu.sync_copy(acc_v, part_hbm.at[pl.ds(base, PW)])


def _sc_combine_body(prev_hbm, lse0_hbm, part_hbm, rowlse_hbm, out_hbm,
                     prev_v, lse_v, part_v, lse0_v, acc_v, sem):
    wid = lax.axis_index("s") * NC + lax.axis_index("c")
    base = wid * PW

    pltpu.sync_copy(prev_hbm.at[wid], prev_v)
    pltpu.sync_copy(part_hbm.at[pl.ds(base, PW)], part_v)
    pltpu.sync_copy(lse0_hbm, lse0_v)
    pltpu.sync_copy(rowlse_hbm, lse_v)

    for cc in range(PW // L):
        sl = pl.ds(cc * L, L)

        def acc_body(j, acc):
            return acc + plsc.load_gather(lse_v, [prev_v[j, sl]])

        acc16 = lax.fori_loop(0, T1, acc_body, jnp.zeros((L,), jnp.float32))
        acc_v[sl] = part_v[sl] - acc16 - lse0_v[...]

    pltpu.sync_copy(acc_v, out_hbm.at[pl.ds(base, PW)])


def _sc_mesh():
    return plsc.VectorSubcoreMesh(
        core_axis_name="c", subcore_axis_name="s", num_cores=NC,
        num_subcores=NS)


@functools.cache
def _sc_gather():
    return pl.kernel(
        _sc_gather_body,
        out_type=jax.ShapeDtypeStruct((B,), jnp.float32),
        mesh=_sc_mesh(),
        scratch_types=[
            pltpu.VMEM((T1, PW), jnp.int32),
            pltpu.VMEM((T1, PW), jnp.int32),
            pltpu.VMEM((T1, PW), jnp.int32),
            pltpu.VMEM((T1, PW), jnp.float32),
            pltpu.VMEM((S,), jnp.float32),
            pltpu.VMEM((PW,), jnp.int32),
            pltpu.VMEM((PW,), jnp.float32),
            pltpu.SemaphoreType.DMA,
        ],
        compiler_params=pltpu.CompilerParams(needs_layout_passes=False),
    )


@functools.cache
def _sc_combine():
    return pl.kernel(
        _sc_combine_body,
        out_type=jax.ShapeDtypeStruct((B,), jnp.float32),
        mesh=_sc_mesh(),
        scratch_types=[
            pltpu.VMEM((T1, PW), jnp.int32),
            pltpu.VMEM((S,), jnp.float32),
            pltpu.VMEM((PW,), jnp.float32),
            pltpu.VMEM((L,), jnp.float32),
            pltpu.VMEM((PW,), jnp.float32),
            pltpu.SemaphoreType.DMA,
        ],
        compiler_params=pltpu.CompilerParams(needs_layout_passes=False),
    )


def kernel(x, initial_state_vector, state_transition_matrix):
    x = x.astype(jnp.int32)
    row_lse, init_lse = _tc_lse(state_transition_matrix, initial_state_vector)

    xt = x.T
    prev_w = xt[:-1].reshape(T1, NW, PW).transpose(1, 0, 2)
    cur_w = xt[1:].reshape(T1, NW, PW).transpose(1, 0, 2)
    first = x[:, 0]
    lse0_vec = jnp.full((L,), init_lse[0, 0], dtype=jnp.float32)
    t4 = state_transition_matrix.reshape(
        S // 8, 8, S // 128, 128).transpose(0, 2, 1, 3)
    t_flat = t4.reshape(-1)

    if N_SC > 0:
        s_sc = _sc_rowsum()(t4)
    partial = _sc_gather()(prev_w, cur_w, first, t_flat,
                           initial_state_vector)
    row_lse_full = _tc_fixup(row_lse, s_sc) if N_SC > 0 else row_lse
    return _sc_combine()(prev_w, lse0_vec, partial, row_lse_full)


# speedup vs baseline: 1.0599x; 1.0136x over previous
import functools

import jax
import jax.numpy as jnp
from jax import lax
from jax.experimental import pallas as pl
from jax.experimental.pallas import tpu as pltpu
from jax.experimental.pallas import tpu_sc as plsc

S = 8192
B = 4096
T = 50
T1 = T - 1
NC = 2
NS = 16
NW = NC * NS
PW = B // NW
L = 16
RBLK = 512
R_TC = 8192
N_SC = S - R_TC
P_SC = N_SC // NW
G_SC = P_SC // 8


def _tc_lse_body(t_ref, init_ref, rowlse_ref, initlse_ref):
    i = pl.program_id(0)
    blk = t_ref[...]
    m = jnp.max(blk, axis=1)
    ssum = jnp.sum(jnp.exp(blk - m[:, None]), axis=1)
    rowlse_ref[...] = m + jnp.log(ssum)

    @pl.when(i == 0)
    def _():
        v = init_ref[...]
        mi = jnp.max(v)
        lse0 = mi + jnp.log(jnp.sum(jnp.exp(v - mi)))
        initlse_ref[...] = jnp.full((8, 128), lse0, dtype=jnp.float32)


def _tc_lse(t_mat, init_vec):
    return pl.pallas_call(
        _tc_lse_body,
        grid=(R_TC // RBLK,),
        in_specs=[
            pl.BlockSpec((RBLK, S), lambda i: (i, 0)),
            pl.BlockSpec((S,), lambda i: (0,)),
        ],
        out_specs=[
            pl.BlockSpec((RBLK,), lambda i: (i,)),
            pl.BlockSpec((8, 128), lambda i: (0, 0)),
        ],
        out_shape=[
            jax.ShapeDtypeStruct((R_TC,), jnp.float32),
            jax.ShapeDtypeStruct((8, 128), jnp.float32),
        ],
        compiler_params=pltpu.CompilerParams(
            dimension_semantics=("arbitrary",),
        ),
    )(t_mat, init_vec)


def _tc_fixup_body(lse_tc_ref, s_sc_ref, out_ref):
    out_ref[pl.ds(0, R_TC)] = lse_tc_ref[...]
    out_ref[pl.ds(R_TC, N_SC)] = jnp.log(s_sc_ref[...])


def _tc_fixup(lse_tc, s_sc):
    return pl.pallas_call(
        _tc_fixup_body,
        out_shape=jax.ShapeDtypeStruct((S,), jnp.float32),
    )(lse_tc, s_sc)


def _sc_rowsum_body(t4_hbm, s_hbm, buf_v, out_v, sems):
    wid = lax.axis_index("s") * NC + lax.axis_index("c")
    rt0 = R_TC // 8 + wid * G_SC

    nch = G_SC * 2
    hct = (S // 128) // 2

    def issue(i):
        g, half = divmod(i, 2)
        return pltpu.async_copy(
            t4_hbm.at[rt0 + g, pl.ds(half * hct, hct)], buf_v.at[i % 2],
            sems.at[i % 2])

    lanes = lax.iota(jnp.int32, L)
    copies = {0: issue(0)}
    rowvec = jnp.zeros((L,), jnp.float32)
    accs = None
    for i in range(nch):
        g, half = divmod(i, 2)
        if i + 1 < nch:
            copies[i + 1] = issue(i + 1)
        copies[i].wait()
        if half == 0:
            accs = tuple(jnp.zeros((L,), jnp.float32) for _ in range(8))

        def ct_body(ct, a8, _i=i):
            new = []
            for sl in range(8):
                a = a8[sl]
                for k in range(8):
                    a = a + jnp.exp(buf_v[_i % 2, ct, sl, pl.ds(k * L, L)])
                new.append(a)
            return tuple(new)

        accs = lax.fori_loop(0, hct, ct_body, accs)
        if half == 1:
            for sl in range(8):
                rowvec = jnp.where(lanes == (g % 2) * 8 + sl,
                                   jnp.sum(accs[sl]), rowvec)
            if g % 2 == 1:
                out_v[pl.ds((g // 2) * L, L)] = rowvec

    pltpu.sync_copy(out_v, s_hbm.at[pl.ds(wid * P_SC, P_SC)])


@functools.cache
def _sc_rowsum():
    return pl.kernel(
        _sc_rowsum_body,
        out_type=jax.ShapeDtypeStruct((N_SC,), jnp.float32),
        mesh=_sc_mesh(),
        scratch_types=[
            pltpu.VMEM((2, (S // 128) // 2, 8, 128), jnp.float32),
            pltpu.VMEM((P_SC,), jnp.float32),
            pltpu.SemaphoreType.DMA((2,)),
        ],
        compiler_params=pltpu.CompilerParams(needs_layout_passes=False),
    )


def _sc_gather_body(prev_hbm, cur_hbm, first_hbm, tflat_hbm, init_hbm,
                    part_hbm,
                    prev_v, cur_v, idx_v, vals_v, init_v, first_v, acc_v, sem):
    wid = lax.axis_index("s") * NC + lax.axis_index("c")
    base = wid * PW

    pltpu.sync_copy(prev_hbm.at[wid], prev_v)
    pltpu.sync_copy(cur_hbm.at[wid], cur_v)
    pltpu.sync_copy(first_hbm.at[pl.ds(base, PW)], first_v)
    pltpu.sync_copy(init_hbm, init_v)

    def idx_body(j, carry):
        for cc in range(PW // L):
            p = prev_v[j, pl.ds(cc * L, L)]
            q = cur_v[j, pl.ds(cc * L, L)]
            idx_v[j, pl.ds(cc * L, L)] = (
                ((p >> 3) << 16) + ((q >> 7) << 10) + ((p & 7) << 7)
                + (q & 127))
        return carry

    lax.fori_loop(0, T1, idx_body, 0)

    def gather_group(g, carry):
        copies = []
        for u in range(7):
            j = g * 7 + u
            copies.append(
                pltpu.async_copy(tflat_hbm.at[idx_v.at[j]], vals_v.at[j], sem))
        for cp in copies:
            cp.wait()
        return carry

    lax.fori_loop(0, 7, gather_group, 0)

    for cc in range(PW // L):
        sl = pl.ds(cc * L, L)

        def acc_body(j, acc):
            return acc + vals_v[j, sl]

        acc16 = lax.fori_loop(0, T1, acc_body, jnp.zeros((L,), jnp.float32))
        acc_v[sl] = acc16 + plsc.load_gather(init_v, [first_v[sl]])

    pltpu.sync_copy(acc_v, part_hbm.at[pl.ds(base, PW)])


def _sc_combine_body(prev_hbm, lse0_hbm, part_hbm, rowlse_hbm, out_hbm,
                     prev_v, lse_v, part_v, lse0_v, acc_v, sem):
    wid = lax.axis_index("s") * NC + lax.axis_index("c")
    base = wid * PW

    pltpu.sync_copy(prev_hbm.at[wid], prev_v)
    pltpu.sync_copy(part_hbm.at[pl.ds(base, PW)], part_v)
    pltpu.sync_copy(lse0_hbm, lse0_v)
    pltpu.sync_copy(rowlse_hbm, lse_v)

    for cc in range(PW // L):
        sl = pl.ds(cc * L, L)

        def acc_body(j, accs):
            return tuple(
                accs[u] + plsc.load_gather(lse_v, [prev_v[j + 12 * u, sl]])
                for u in range(4))

        a = lax.fori_loop(
            0, 12, acc_body,
            tuple(jnp.zeros((L,), jnp.float32) for _ in range(4)))
        acc16 = ((a[0] + a[1]) + (a[2] + a[3])
                 + plsc.load_gather(lse_v, [prev_v[48, sl]]))
        acc_v[sl] = part_v[sl] - acc16 - lse0_v[...]

    pltpu.sync_copy(acc_v, out_hbm.at[pl.ds(base, PW)])


def _sc_mesh():
    return plsc.VectorSubcoreMesh(
        core_axis_name="c", subcore_axis_name="s", num_cores=NC,
        num_subcores=NS)


@functools.cache
def _sc_gather():
    return pl.kernel(
        _sc_gather_body,
        out_type=jax.ShapeDtypeStruct((B,), jnp.float32),
        mesh=_sc_mesh(),
        scratch_types=[
            pltpu.VMEM((T1, PW), jnp.int32),
            pltpu.VMEM((T1, PW), jnp.int32),
            pltpu.VMEM((T1, PW), jnp.int32),
            pltpu.VMEM((T1, PW), jnp.float32),
            pltpu.VMEM((S,), jnp.float32),
            pltpu.VMEM((PW,), jnp.int32),
            pltpu.VMEM((PW,), jnp.float32),
            pltpu.SemaphoreType.DMA,
        ],
        compiler_params=pltpu.CompilerParams(needs_layout_passes=False),
    )


@functools.cache
def _sc_combine():
    return pl.kernel(
        _sc_combine_body,
        out_type=jax.ShapeDtypeStruct((B,), jnp.float32),
        mesh=_sc_mesh(),
        scratch_types=[
            pltpu.VMEM((T1, PW), jnp.int32),
            pltpu.VMEM((S,), jnp.float32),
            pltpu.VMEM((PW,), jnp.float32),
            pltpu.VMEM((L,), jnp.float32),
            pltpu.VMEM((PW,), jnp.float32),
            pltpu.SemaphoreType.DMA,
        ],
        compiler_params=pltpu.CompilerParams(needs_layout_passes=False),
    )


def kernel(x, initial_state_vector, state_transition_matrix):
    x = x.astype(jnp.int32)
    row_lse, init_lse = _tc_lse(state_transition_matrix, initial_state_vector)

    xt = x.T
    prev_w = xt[:-1].reshape(T1, NW, PW).transpose(1, 0, 2)
    cur_w = xt[1:].reshape(T1, NW, PW).transpose(1, 0, 2)
    first = x[:, 0]
    lse0_vec = jnp.full((L,), init_lse[0, 0], dtype=jnp.float32)
    t4 = state_transition_matrix.reshape(
        S // 8, 8, S // 128, 128).transpose(0, 2, 1, 3)
    t_flat = t4.reshape(-1)

    if N_SC > 0:
        s_sc = _sc_rowsum()(t4)
    partial = _sc_gather()(prev_w, cur_w, first, t_flat,
                           initial_state_vector)
    row_lse_full = _tc_fixup(row_lse, s_sc) if N_SC > 0 else row_lse
    return _sc_combine()(prev_w, lse0_vec, partial, row_lse_full)
